# Initial kernel scaffold; baseline (speedup 1.0000x reference)
import functools

import jax
import jax.numpy as jnp
from jax import lax
from jax.experimental import pallas as pl
from jax.experimental.pallas import tpu as pltpu
from jax.experimental.pallas import tpu_sc as plsc


def _build(B, T, D, vocab):
    info = plsc.get_sparse_core_info()
    NC, NS = info.num_cores, info.num_subcores
    NW = NC * NS
    BT = B * T
    assert BT % NW == 0
    rows_w = BT // NW
    assert rows_w % T == 0
    assert T % 2 == 0
    CW = T // 2
    assert CW <= 128
    n_chunks = rows_w // CW

    mesh = plsc.VectorSubcoreMesh(core_axis_name="c", subcore_axis_name="s")

    @functools.partial(
        pl.kernel,
        out_type=jax.ShapeDtypeStruct((BT, D), jnp.float32),
        mesh=mesh,
        scratch_types=[
            pltpu.VMEM((n_chunks, CW), jnp.int32),
            pltpu.VMEM((T, D), jnp.float32),
            pltpu.VMEM((CW, D), jnp.float32),
            pltpu.VMEM((CW, D), jnp.float32),
            pltpu.SemaphoreType.DMA,
            pltpu.SemaphoreType.DMA,
        ],
    )
    def enc(obs_hbm, embed_hbm, pos_hbm, out_hbm, idx_v, pos_v, buf0, buf1,
            gsem, osem):
        wid = lax.axis_index("s") * NC + lax.axis_index("c")
        base = wid * rows_w
        pltpu.sync_copy(obs_hbm.at[wid], idx_v)
        pltpu.sync_copy(pos_hbm, pos_v)

        def body(i, carry):
            for k, buf in ((0, buf0), (1, buf1)):
                c = 2 * i + k
                pltpu.sync_copy(pos_v.at[pl.ds(k * CW, CW)], buf)
                pltpu.async_copy(embed_hbm.at[idx_v.at[c]], buf, gsem,
                                 add=True).wait()
                pltpu.async_copy(buf, out_hbm.at[pl.ds(base + c * CW, CW)],
                                 osem).wait()
            return carry

        lax.fori_loop(0, n_chunks // 2, body, 0)

    return enc


def kernel(obs, embed, pos_embed):
    B, T = obs.shape
    vocab, D = embed.shape
    enc = _build(B, T, D, vocab)
    info = plsc.get_sparse_core_info()
    NW = info.num_cores * info.num_subcores
    CW = T // 2
    obs_i = obs.astype(jnp.int32).reshape(NW, (B * T) // (NW * CW), CW)
    out = enc(obs_i, embed, pos_embed[:T])
    return out.reshape(B, T, D)


# speedup vs baseline: 3.4921x; 3.4921x over previous
import functools

import jax
import jax.numpy as jnp
from jax import lax
from jax.experimental import pallas as pl
from jax.experimental.pallas import tpu as pltpu
from jax.experimental.pallas import tpu_sc as plsc


def _build(B, T, D, vocab):
    info = plsc.get_sparse_core_info()
    NC, NS = info.num_cores, info.num_subcores
    NW = NC * NS
    BT = B * T
    assert BT % NW == 0
    rows_w = BT // NW
    assert rows_w % T == 0
    assert T % 2 == 0
    CW = T // 2
    assert CW <= 128
    n_chunks = rows_w // CW

    mesh = plsc.VectorSubcoreMesh(core_axis_name="c", subcore_axis_name="s")

    @functools.partial(
        pl.kernel,
        out_type=jax.ShapeDtypeStruct((BT, D), jnp.float32),
        mesh=mesh,
        compiler_params=pltpu.CompilerParams(use_tc_tiling_on_sc=False),
        scratch_types=[
            pltpu.VMEM((n_chunks, CW), jnp.int32),
            pltpu.VMEM_SHARED((T, D), jnp.float32),
            pltpu.VMEM((T, D), jnp.float32),
            pltpu.SemaphoreType.DMA,
            pltpu.SemaphoreType.DMA,
        ],
    )
    def enc(obs_hbm, embed_hbm, pos_hbm, out_hbm, idx_v, pos_sh, buf,
            gsem, osem):
        sid = lax.axis_index("s")
        wid = sid * NC + lax.axis_index("c")
        base = wid * rows_w
        pltpu.sync_copy(obs_hbm.at[wid], idx_v)
        @pl.when(sid == 0)
        def _():
            pltpu.sync_copy(pos_hbm, pos_sh)
        plsc.subcore_barrier()
        n_seq = rows_w // T

        def body(s, carry):
            pltpu.sync_copy(pos_sh, buf)
            d0 = pltpu.async_copy(embed_hbm.at[idx_v.at[2 * s]],
                                  buf.at[pl.ds(0, CW)], gsem, add=True)
            d1 = pltpu.async_copy(embed_hbm.at[idx_v.at[2 * s + 1]],
                                  buf.at[pl.ds(CW, CW)], gsem, add=True)
            d0.wait()
            d1.wait()
            pltpu.async_copy(buf, out_hbm.at[pl.ds(base + s * T, T)],
                             osem).wait()
            return carry

        lax.fori_loop(0, n_seq, body, 0)

    return enc


def kernel(obs, embed, pos_embed):
    B, T = obs.shape
    vocab, D = embed.shape
    enc = _build(B, T, D, vocab)
    info = plsc.get_sparse_core_info()
    NW = info.num_cores * info.num_subcores
    CW = T // 2
    obs_i = obs.astype(jnp.int32).reshape(NW, (B * T) // (NW * CW), CW)
    out = enc(obs_i, embed, pos_embed[:T])
    return out.reshape(B, T, D)


# speedup vs baseline: 3.9912x; 1.1429x over previous
import functools

import jax
import jax.numpy as jnp
from jax import lax
from jax.experimental import pallas as pl
from jax.experimental.pallas import tpu as pltpu
from jax.experimental.pallas import tpu_sc as plsc

NBUF = 4


def _build(B, T, D, vocab):
    info = plsc.get_sparse_core_info()
    NC, NS = info.num_cores, info.num_subcores
    NW = NC * NS
    BT = B * T
    assert BT % NW == 0
    rows_w = BT // NW
    assert rows_w % T == 0
    assert T % 2 == 0 and T % 8 == 0
    CW = T // 2
    assert CW <= 128
    n_chunks = rows_w // CW
    n_seq = rows_w // T
    assert n_seq % NBUF == 0

    mesh = plsc.VectorSubcoreMesh(core_axis_name="c", subcore_axis_name="s")

    @functools.partial(
        pl.kernel,
        out_type=jax.ShapeDtypeStruct((BT, D), jnp.float32),
        mesh=mesh,
        compiler_params=pltpu.CompilerParams(use_tc_tiling_on_sc=False),
        scratch_types=(
            [pltpu.VMEM((n_chunks, CW), jnp.int32)]
            + [pltpu.VMEM_SHARED((T, D), jnp.float32)]
            + [pltpu.VMEM((T, D), jnp.float32)] * NBUF
            + [pltpu.SemaphoreType.DMA] * (3 * NBUF)
        ),
    )
    def enc(obs_hbm, embed_hbm, pos_hbm, out_hbm, idx_v, pos_sh, *rest):
        bufs = rest[:NBUF]
        ssem = rest[NBUF:2 * NBUF]
        gsem = rest[2 * NBUF:3 * NBUF]
        osem = rest[3 * NBUF:4 * NBUF]
        sid = lax.axis_index("s")
        wid = sid * NC + lax.axis_index("c")
        base = wid * rows_w
        pltpu.sync_copy(obs_hbm.at[wid], idx_v)
        @pl.when(sid == 0)
        def _():
            pltpu.sync_copy(pos_hbm, pos_sh)
        plsc.subcore_barrier()

        pltpu.async_copy(pos_sh, bufs[0], ssem[0])
        pltpu.async_copy(pos_sh, bufs[1], ssem[1])

        def visit(s, b):
            pltpu.make_async_copy(pos_sh, bufs[b], ssem[b]).wait()
            d0 = pltpu.async_copy(embed_hbm.at[idx_v.at[2 * s]],
                                  bufs[b].at[pl.ds(0, CW)], gsem[b], add=True)
            d1 = pltpu.async_copy(embed_hbm.at[idx_v.at[2 * s + 1]],
                                  bufs[b].at[pl.ds(CW, CW)], gsem[b], add=True)
            b2 = (b + 2) % NBUF
            @pl.when(s + 2 < n_seq)
            def _():
                @pl.when(s >= 2)
                def _():
                    pltpu.make_async_copy(
                        bufs[b2], out_hbm.at[pl.ds(base, T)], osem[b2]).wait()
                pltpu.async_copy(pos_sh, bufs[b2], ssem[b2])
            d0.wait()
            d1.wait()
            pltpu.async_copy(bufs[b], out_hbm.at[pl.ds(base + s * T, T)],
                             osem[b])

        def body(g, carry):
            for b in range(NBUF):
                visit(NBUF * g + b, b)
            return carry

        lax.fori_loop(0, n_seq // NBUF, body, 0)
        for b in range(NBUF):
            pltpu.make_async_copy(
                bufs[b], out_hbm.at[pl.ds(base, T)], osem[b]).wait()

    return enc


def kernel(obs, embed, pos_embed):
    B, T = obs.shape
    vocab, D = embed.shape
    enc = _build(B, T, D, vocab)
    info = plsc.get_sparse_core_info()
    NW = info.num_cores * info.num_subcores
    CW = T // 2
    obs_i = obs.astype(jnp.int32).reshape(NW, (B * T) // (NW * CW), CW)
    out = enc(obs_i, embed, pos_embed[:T])
    return out.reshape(B, T, D)


# speedup vs baseline: 3.9986x; 1.0019x over previous
import functools

import jax
import jax.numpy as jnp
from jax import lax
from jax.experimental import pallas as pl
from jax.experimental.pallas import tpu as pltpu
from jax.experimental.pallas import tpu_sc as plsc

NBUF = 4


def _build(B, T, D, vocab):
    info = plsc.get_sparse_core_info()
    NC, NS = info.num_cores, info.num_subcores
    NW = NC * NS
    assert B % NW == 0
    n_b = B // NW
    assert T % 8 == 0
    CW0 = min(T, 128)
    CW1 = T - CW0
    assert CW1 <= 128 and CW0 % 8 == 0 and CW1 % 8 == 0
    assert n_b % NBUF == 0

    mesh = plsc.VectorSubcoreMesh(core_axis_name="c", subcore_axis_name="s")

    @functools.partial(
        pl.kernel,
        out_type=jax.ShapeDtypeStruct((B, T, D), jnp.float32),
        mesh=mesh,
        compiler_params=pltpu.CompilerParams(use_tc_tiling_on_sc=False),
        scratch_types=(
            [pltpu.VMEM((n_b, T), jnp.int32)]
            + [pltpu.VMEM_SHARED((T, D), jnp.float32)]
            + [pltpu.VMEM((T, D), jnp.float32)] * NBUF
            + [pltpu.SemaphoreType.DMA] * (3 * NBUF)
        ),
    )
    def enc(obs_hbm, embed_hbm, pos_hbm, out_hbm, idx_v, pos_sh, *rest):
        bufs = rest[:NBUF]
        ssem = rest[NBUF:2 * NBUF]
        gsem = rest[2 * NBUF:3 * NBUF]
        osem = rest[3 * NBUF:4 * NBUF]
        sid = lax.axis_index("s")
        wid = sid * NC + lax.axis_index("c")
        base = wid * n_b
        pltpu.sync_copy(obs_hbm.at[pl.ds(base, n_b)], idx_v)
        @pl.when(sid == 0)
        def _():
            pltpu.sync_copy(pos_hbm.at[pl.ds(0, T)], pos_sh)
        plsc.subcore_barrier()

        pltpu.async_copy(pos_sh, bufs[0], ssem[0])
        pltpu.async_copy(pos_sh, bufs[1], ssem[1])

        def visit(s, b):
            pltpu.make_async_copy(pos_sh, bufs[b], ssem[b]).wait()
            d0 = pltpu.async_copy(embed_hbm.at[idx_v.at[s, pl.ds(0, CW0)]],
                                  bufs[b].at[pl.ds(0, CW0)], gsem[b], add=True)
            d1 = pltpu.async_copy(embed_hbm.at[idx_v.at[s, pl.ds(CW0, CW1)]],
                                  bufs[b].at[pl.ds(CW0, CW1)], gsem[b],
                                  add=True)
            b2 = (b + 2) % NBUF
            @pl.when(s + 2 < n_b)
            def _():
                @pl.when(s >= 2)
                def _():
                    pltpu.make_async_copy(
                        bufs[b2], out_hbm.at[base], osem[b2]).wait()
                pltpu.async_copy(pos_sh, bufs[b2], ssem[b2])
            d0.wait()
            d1.wait()
            pltpu.async_copy(bufs[b], out_hbm.at[base + s], osem[b])

        def body(g, carry):
            for b in range(NBUF):
                visit(NBUF * g + b, b)
            return carry

        lax.fori_loop(0, n_b // NBUF, body, 0)
        for b in range(NBUF):
            pltpu.make_async_copy(bufs[b], out_hbm.at[base], osem[b]).wait()

    return enc


def kernel(obs, embed, pos_embed):
    B, T = obs.shape
    vocab, D = embed.shape
    enc = _build(B, T, D, vocab)
    return enc(obs.astype(jnp.int32), embed, pos_embed)


# speedup vs baseline: 4.8908x; 1.2231x over previous
import functools

import jax
import jax.numpy as jnp
from jax import lax
from jax.experimental import pallas as pl
from jax.experimental.pallas import tpu as pltpu
from jax.experimental.pallas import tpu_sc as plsc

NBUF = 4
CW = 128
LW = 128


def _build(B, T, D, vocab):
    info = plsc.get_sparse_core_info()
    NC, NS = info.num_cores, info.num_subcores
    NW = NC * NS
    BT = B * T
    assert BT % (NW * CW) == 0
    rows_w = BT // NW
    n_chunks = rows_w // CW
    assert rows_w % T == 0
    assert (CW % 8 == 0) and (T % 8 == 0)
    assert n_chunks % NBUF == 0

    mesh = plsc.VectorSubcoreMesh(core_axis_name="c", subcore_axis_name="s")

    @functools.partial(
        pl.kernel,
        out_type=jax.ShapeDtypeStruct((BT, LW), jnp.float32),
        mesh=mesh,
        compiler_params=pltpu.CompilerParams(use_tc_tiling_on_sc=True),
        scratch_types=(
            [pltpu.VMEM((n_chunks, CW), jnp.int32)]
            + [pltpu.VMEM_SHARED((T + CW, LW), jnp.float32)]
            + [pltpu.VMEM((CW, LW), jnp.float32)] * NBUF
            + [pltpu.SemaphoreType.DMA] * (3 * NBUF)
        ),
    )
    def enc(obs_hbm, embed_hbm, pos_hbm, out_hbm, idx_v, pos_sh, *rest):
        bufs = rest[:NBUF]
        ssem = rest[NBUF:2 * NBUF]
        gsem = rest[2 * NBUF:3 * NBUF]
        osem = rest[3 * NBUF:4 * NBUF]
        sid = lax.axis_index("s")
        wid = sid * NC + lax.axis_index("c")
        base = wid * rows_w
        pltpu.sync_copy(obs_hbm.at[wid], idx_v)
        @pl.when(sid == 0)
        def _():
            pltpu.sync_copy(pos_hbm, pos_sh)
        plsc.subcore_barrier()

        def seed(s, b):
            off = (s * CW) % T
            pltpu.async_copy(pos_sh.at[pl.ds(off, CW)], bufs[b], ssem[b])

        seed(0, 0)
        seed(1, 1)

        def visit(s, b):
            pltpu.make_async_copy(pos_sh.at[pl.ds(0, CW)], bufs[b],
                                  ssem[b]).wait()
            d0 = pltpu.async_copy(embed_hbm.at[idx_v.at[s]], bufs[b],
                                  gsem[b], add=True)
            b2 = (b + 2) % NBUF
            @pl.when(s + 2 < n_chunks)
            def _():
                @pl.when(s >= 2)
                def _():
                    pltpu.make_async_copy(
                        bufs[b2], out_hbm.at[pl.ds(base, CW)],
                        osem[b2]).wait()
                seed(s + 2, b2)
            d0.wait()
            pltpu.async_copy(bufs[b], out_hbm.at[pl.ds(base + s * CW, CW)],
                             osem[b])

        def body(g, carry):
            for b in range(NBUF):
                visit(NBUF * g + b, b)
            return carry

        lax.fori_loop(0, n_chunks // NBUF, body, 0)
        for b in range(NBUF):
            pltpu.make_async_copy(bufs[b], out_hbm.at[pl.ds(base, CW)],
                                  osem[b]).wait()

    return enc


def kernel(obs, embed, pos_embed):
    B, T = obs.shape
    vocab, D = embed.shape
    enc = _build(B, T, D, vocab)
    info = plsc.get_sparse_core_info()
    NW = info.num_cores * info.num_subcores
    obs_i = obs.astype(jnp.int32).reshape(NW, (B * T) // (NW * CW), CW)
    embed_p = jnp.pad(embed, ((0, 0), (0, LW - D)))
    pos_p = jnp.pad(pos_embed[:T], ((0, 0), (0, LW - D)))
    pos_w = jnp.concatenate([pos_p, pos_p[:CW]], axis=0)
    out = enc(obs_i, embed_p, pos_w)
    return out[:, :D].reshape(B, T, D)
